# baseline (speedup 1.0000x reference)
import functools
import math

import jax
import jax.numpy as jnp
from jax import lax
from jax.experimental import pallas as pl
from jax.experimental.pallas import tpu as pltpu
from jax.experimental.pallas import tpu_sc as plsc

D_MODEL = 64
VOCAB = 1000000
SCALE = math.sqrt(D_MODEL)

NC = 2
NS = 16
NW = NC * NS
LANES = 16

NI = 4096
NJ = 200
CH = 128
NUNITS = NJ * (NI // CH)
PER_W = NUNITS // NW
NBUF = 4

NSLAB = VOCAB // CH
SLAB_MAX = NSLAB // NW + 1
TAIL_C0 = NSLAB * CH
TAIL_W = VOCAB - TAIL_C0

_params = pltpu.CompilerParams(use_tc_tiling_on_sc=True, needs_layout_passes=False)


def _sc_repack(lutT, tail2):
    mesh = plsc.VectorSubcoreMesh(core_axis_name="c", subcore_axis_name="s")

    @functools.partial(
        pl.kernel,
        mesh=mesh,
        out_type=jax.ShapeDtypeStruct((VOCAB // 2, 2 * D_MODEL), jnp.float32),
        scratch_types=[
            pltpu.VMEM((D_MODEL, CH), jnp.float32),
            pltpu.VMEM((D_MODEL, CH), jnp.float32),
            pltpu.VMEM((D_MODEL, 2 * D_MODEL), jnp.float32),
            pltpu.SemaphoreType.DMA,
            pltpu.SemaphoreType.DMA,
        ],
        compiler_params=_params,
    )
    def k(lutT_hbm, tail2_hbm, t2_hbm, slab0, slab1, obuf, sem0, sem1):
        slabs = (slab0, slab1)
        sems = (sem0, sem1)
        cid = lax.axis_index("c")
        sid = lax.axis_index("s")
        wid = cid * NS + sid

        @pl.when(wid < NSLAB)
        def _():
            pltpu.async_copy(lutT_hbm.at[:, pl.ds(wid * CH, CH)], slabs[0], sems[0])

        riota = lax.iota(jnp.int32, LANES)

        def trip(g, carry):
            for b in range(2):
                s = wid + (g * 2 + b) * NW

                @pl.when(s < NSLAB)
                def _(b=b, s=s):
                    pltpu.make_async_copy(
                        lutT_hbm.at[:, pl.ds(s * CH, CH)], slabs[b], sems[b]
                    ).wait()
                    s2 = s + NW

                    @pl.when(s2 < NSLAB)
                    def _(b=b, s2=s2):
                        pltpu.async_copy(
                            lutT_hbm.at[:, pl.ds(s2 * CH, CH)],
                            slabs[1 - b], sems[1 - b]
                        )

                    def qloop(q, c, b=b):
                        c0 = 2 * q
                        c1 = c0 + 1
                        for k0 in range(2 * D_MODEL // LANES):
                            col = c0 if k0 < D_MODEL // LANES else c1
                            rvec = riota + (k0 % (D_MODEL // LANES)) * LANES
                            cvec = jnp.broadcast_to(col, (LANES,)).astype(jnp.int32)
                            v = plsc.load_gather(slabs[b], [rvec, cvec])
                            obuf[q, pl.ds(k0 * LANES, LANES)] = v
                        return c

                    lax.fori_loop(0, D_MODEL, qloop, 0, unroll=2)
                    pltpu.sync_copy(obuf, t2_hbm.at[pl.ds(s * D_MODEL, D_MODEL)])

            return carry

        lax.fori_loop(0, (SLAB_MAX + 1) // 2, trip, 0)

        @pl.when(wid == NW - 1)
        def _():
            pltpu.sync_copy(tail2_hbm, obuf.at[pl.ds(0, TAIL_W // 2)])
            pltpu.sync_copy(
                obuf.at[pl.ds(0, TAIL_W // 2)],
                t2_hbm.at[pl.ds(TAIL_C0 // 2, TAIL_W // 2)],
            )

    return k(lutT, tail2)


def _sc_gather(xT, t2):
    mesh = plsc.VectorSubcoreMesh(core_axis_name="c", subcore_axis_name="s")

    @functools.partial(
        pl.kernel,
        mesh=mesh,
        out_type=jax.ShapeDtypeStruct((NJ, D_MODEL, NI), jnp.float32),
        scratch_types=[
            *[pltpu.VMEM((CH,), jnp.int32) for _ in range(NBUF)],
            *[pltpu.VMEM((CH,), jnp.int32) for _ in range(NBUF)],
            *[pltpu.VMEM((CH, 2 * D_MODEL), jnp.float32) for _ in range(NBUF)],
            pltpu.VMEM((D_MODEL, CH), jnp.float32),
            *[pltpu.SemaphoreType.DMA for _ in range(NBUF)],
        ],
        compiler_params=_params,
    )
    def k(xT_hbm, t2_hbm, o3_hbm, *scr):
        idxb = scr[:NBUF]
        pidb = scr[NBUF:2 * NBUF]
        gb = scr[2 * NBUF:3 * NBUF]
        tbuf = scr[3 * NBUF]
        sems = scr[3 * NBUF + 1:]

        cid = lax.axis_index("c")
        sid = lax.axis_index("s")
        wid = cid * NS + sid
        u0 = wid * PER_W

        def fetch(u, b):
            j = lax.shift_right_logical(u, 5)
            i0 = lax.mul(lax.bitwise_and(u, 31), CH)
            pltpu.sync_copy(xT_hbm.at[j, pl.ds(i0, CH)], idxb[b])
            for t in range(CH // LANES):
                sl = pl.ds(t * LANES, LANES)
                pidb[b][sl] = lax.shift_right_logical(idxb[b][sl], 1)
            pltpu.async_copy(t2_hbm.at[pidb[b]], gb[b], sems[b])

        for b in range(NBUF):
            fetch(u0 + b, b)

        riota = lax.iota(jnp.int32, LANES)

        def step(g, carry):
            for b in range(NBUF):
                u = u0 + g * NBUF + b
                j = lax.shift_right_logical(u, 5)
                i0 = lax.mul(lax.bitwise_and(u, 31), CH)
                pltpu.make_async_copy(t2_hbm.at[pidb[b]], gb[b], sems[b]).wait()

                for t in range(CH // LANES):
                    r0 = t * LANES
                    sl = pl.ds(r0, LANES)
                    hv = lax.mul(lax.bitwise_and(idxb[b][sl], 1), D_MODEL)
                    rows = riota + r0

                    def dloop(d, c, hv=hv, rows=rows, b=b, r0=r0):
                        v = plsc.load_gather(gb[b], [rows, hv + d]) * SCALE
                        tbuf[d, pl.ds(r0, LANES)] = v
                        return c

                    lax.fori_loop(0, D_MODEL, dloop, 0, unroll=8)

                pltpu.sync_copy(tbuf, o3_hbm.at[j, :, pl.ds(i0, CH)])

                nxt = g * NBUF + b + NBUF

                @pl.when(nxt < PER_W)
                def _(b=b, nxt=nxt):
                    fetch(u0 + nxt, b)

            return carry

        lax.fori_loop(0, PER_W // NBUF, step, 0)

    return k(xT, t2)


def kernel(x, lut):
    xT = x.astype(jnp.int32).T
    lutT = lut.T
    tail2 = lut[TAIL_C0:].reshape(TAIL_W // 2, 2 * D_MODEL)
    t2 = _sc_repack(lutT, tail2)
    o3 = _sc_gather(xT, t2)
    return o3.transpose(2, 0, 1)


# speedup vs baseline: 2.1075x; 2.1075x over previous
import functools
import math

import jax
import jax.numpy as jnp
from jax import lax
from jax.experimental import pallas as pl
from jax.experimental.pallas import tpu as pltpu
from jax.experimental.pallas import tpu_sc as plsc

D_MODEL = 64
VOCAB = 1000000
SCALE = math.sqrt(D_MODEL)

NC = 2
NS = 16
NW = NC * NS
LANES = 16

NI = 4096
NJ = 200
CH = 128
NUNITS = NJ * (NI // CH)
PER_W = NUNITS // NW
NIDX = PER_W * CH
NBUF = 4

NSLAB = VOCAB // CH
SLAB_MAX = NSLAB // NW + 1
TAIL_C0 = NSLAB * CH
TAIL_W = VOCAB - TAIL_C0

_params = pltpu.CompilerParams(use_tc_tiling_on_sc=True, needs_layout_passes=False)


def _sc_repack(lutT, tail2):
    mesh = plsc.VectorSubcoreMesh(core_axis_name="c", subcore_axis_name="s")

    @functools.partial(
        pl.kernel,
        mesh=mesh,
        out_type=jax.ShapeDtypeStruct((VOCAB // 2, 2 * D_MODEL), jnp.float32),
        scratch_types=[
            pltpu.VMEM((D_MODEL, CH), jnp.float32),
            pltpu.VMEM((D_MODEL, CH), jnp.float32),
            pltpu.VMEM((D_MODEL, 2 * D_MODEL), jnp.float32),
            pltpu.VMEM((D_MODEL, 2 * D_MODEL), jnp.float32),
            pltpu.SemaphoreType.DMA,
            pltpu.SemaphoreType.DMA,
            pltpu.SemaphoreType.DMA,
            pltpu.SemaphoreType.DMA,
        ],
        compiler_params=_params,
    )
    def k(lutT_hbm, tail2_hbm, t2_hbm, slab0, slab1, ob0, ob1, sem0, sem1, os0, os1):
        slabs = (slab0, slab1)
        obufs = (ob0, ob1)
        sems = (sem0, sem1)
        osems = (os0, os1)
        cid = lax.axis_index("c")
        sid = lax.axis_index("s")
        wid = cid * NS + sid

        pltpu.async_copy(lutT_hbm.at[:, pl.ds(wid * CH, CH)], slabs[0], sems[0])

        riota = lax.iota(jnp.int32, LANES)

        def trip(g, carry):
            for b in range(2):
                s = wid + (g * 2 + b) * NW

                @pl.when(s < NSLAB)
                def _(b=b, s=s, g=g):
                    pltpu.make_async_copy(
                        lutT_hbm.at[:, pl.ds(s * CH, CH)], slabs[b], sems[b]
                    ).wait()
                    s2 = s + NW

                    @pl.when(s2 < NSLAB)
                    def _(b=b, s2=s2):
                        pltpu.async_copy(
                            lutT_hbm.at[:, pl.ds(s2 * CH, CH)],
                            slabs[1 - b], sems[1 - b]
                        )

                    @pl.when(g * 2 + b >= 2)
                    def _(b=b, s=s):
                        pltpu.make_async_copy(
                            obufs[b],
                            t2_hbm.at[pl.ds((s - 2 * NW) * D_MODEL, D_MODEL)],
                            osems[b],
                        ).wait()

                    @plsc.parallel_loop(0, D_MODEL, 1, unroll=4)
                    def qloop(q, b=b):
                        c0 = 2 * q
                        for k0 in range(2 * D_MODEL // LANES):
                            col = c0 + (0 if k0 < D_MODEL // LANES else 1)
                            rvec = riota + (k0 % (D_MODEL // LANES)) * LANES
                            cvec = jnp.broadcast_to(col, (LANES,)).astype(jnp.int32)
                            v = plsc.load_gather(slabs[b], [rvec, cvec])
                            obufs[b][q, pl.ds(k0 * LANES, LANES)] = v

                    pltpu.async_copy(
                        obufs[b], t2_hbm.at[pl.ds(s * D_MODEL, D_MODEL)], osems[b]
                    )

            return carry

        lax.fori_loop(0, (SLAB_MAX + 1) // 2, trip, 0)

        for b in range(2):
            last = wid + (SLAB_MAX - 1) * NW

            @pl.when(wid + b * NW < NSLAB)
            def _(b=b):
                pltpu.make_async_copy(
                    obufs[b], t2_hbm.at[pl.ds(0, D_MODEL)], osems[b]
                ).wait()

        @pl.when(wid == NW - 1)
        def _():
            pltpu.sync_copy(tail2_hbm, ob0.at[pl.ds(0, TAIL_W // 2)])
            pltpu.sync_copy(
                ob0.at[pl.ds(0, TAIL_W // 2)],
                t2_hbm.at[pl.ds(TAIL_C0 // 2, TAIL_W // 2)],
            )

    return k(lutT, tail2)


def _sc_gather(xflat, t2):
    mesh = plsc.VectorSubcoreMesh(core_axis_name="c", subcore_axis_name="s")

    @functools.partial(
        pl.kernel,
        mesh=mesh,
        out_type=jax.ShapeDtypeStruct((NJ, D_MODEL, NI), jnp.float32),
        scratch_types=[
            pltpu.VMEM((NIDX,), jnp.int32),
            *[pltpu.VMEM((CH,), jnp.int32) for _ in range(NBUF)],
            *[pltpu.VMEM((CH, 2 * D_MODEL), jnp.float32) for _ in range(NBUF)],
            *[pltpu.VMEM((D_MODEL, CH), jnp.float32) for _ in range(2)],
            *[pltpu.SemaphoreType.DMA for _ in range(NBUF)],
            *[pltpu.SemaphoreType.DMA for _ in range(2)],
        ],
        compiler_params=_params,
    )
    def k(x_hbm, t2_hbm, o3_hbm, idxall, *scr):
        pidb = scr[:NBUF]
        gb = scr[NBUF:2 * NBUF]
        tbufs = scr[2 * NBUF:2 * NBUF + 2]
        sems = scr[2 * NBUF + 2:3 * NBUF + 2]
        osems = scr[3 * NBUF + 2:]

        cid = lax.axis_index("c")
        sid = lax.axis_index("s")
        wid = cid * NS + sid
        u0 = wid * PER_W

        pltpu.sync_copy(x_hbm.at[pl.ds(wid * NIDX, NIDX)], idxall)

        def fetch(t, b):
            for q in range(CH // LANES):
                sl = pl.ds(t * CH + q * LANES, LANES)
                pidb[b][pl.ds(q * LANES, LANES)] = lax.shift_right_logical(
                    idxall[sl], 1
                )
            pltpu.async_copy(t2_hbm.at[pidb[b]], gb[b], sems[b])

        for b in range(NBUF):
            fetch(b, b)

        riota = lax.iota(jnp.int32, LANES)

        def step(g, carry):
            for b in range(NBUF):
                t = g * NBUF + b
                u = u0 + t
                j = lax.shift_right_logical(u, 5)
                i0 = lax.mul(lax.bitwise_and(u, 31), CH)
                tb = b % 2
                pltpu.make_async_copy(t2_hbm.at[pidb[b]], gb[b], sems[b]).wait()

                @pl.when(t >= 2)
                def _(tb=tb):
                    pltpu.make_async_copy(
                        tbufs[tb], o3_hbm.at[0, :, pl.ds(0, CH)], osems[tb]
                    ).wait()

                for q in range(CH // LANES):
                    r0 = q * LANES
                    sl = pl.ds(t * CH + r0, LANES)
                    hv = lax.mul(lax.bitwise_and(idxall[sl], 1), D_MODEL)
                    rows = riota + r0

                    @plsc.parallel_loop(0, D_MODEL, 1, unroll=8)
                    def dloop(d, hv=hv, rows=rows, b=b, r0=r0, tb=tb):
                        v = plsc.load_gather(gb[b], [rows, hv + d]) * SCALE
                        tbufs[tb][d, pl.ds(r0, LANES)] = v

                pltpu.async_copy(
                    tbufs[tb], o3_hbm.at[j, :, pl.ds(i0, CH)], osems[tb]
                )

                nxt = t + NBUF

                @pl.when(nxt < PER_W)
                def _(b=b, nxt=nxt):
                    fetch(nxt, b)

            return carry

        lax.fori_loop(0, PER_W // NBUF, step, 0)

        for tb in range(2):
            pltpu.make_async_copy(
                tbufs[tb], o3_hbm.at[0, :, pl.ds(0, CH)], osems[tb]
            ).wait()

    return k(xflat, t2)


def kernel(x, lut):
    xflat = x.astype(jnp.int32).T.reshape(-1)
    lutT = lut.T
    tail2 = lut[TAIL_C0:].reshape(TAIL_W // 2, 2 * D_MODEL)
    t2 = _sc_repack(lutT, tail2)
    o3 = _sc_gather(xflat, t2)
    return o3.transpose(2, 0, 1)


# speedup vs baseline: 2.1163x; 1.0041x over previous
import functools
import math

import jax
import jax.numpy as jnp
from jax import lax
from jax.experimental import pallas as pl
from jax.experimental.pallas import tpu as pltpu
from jax.experimental.pallas import tpu_sc as plsc

D_MODEL = 64
VOCAB = 1000000
SCALE = math.sqrt(D_MODEL)

NC = 2
NS = 16
NW = NC * NS
LANES = 16

NI = 4096
NJ = 200
CH = 128
NUNITS = NJ * (NI // CH)
PER_W = NUNITS // NW
NIDX = PER_W * CH
NBUF = 4

NSLAB = VOCAB // CH
SLAB_MAX = NSLAB // NW + 1
SRING = 4
TAIL_C0 = NSLAB * CH
TAIL_W = VOCAB - TAIL_C0

_tc_tiled = pltpu.CompilerParams(use_tc_tiling_on_sc=True, needs_layout_passes=False)
_linear = pltpu.CompilerParams(use_tc_tiling_on_sc=False, needs_layout_passes=False)


def _sc_repack(lutT, tail2):
    mesh = plsc.VectorSubcoreMesh(core_axis_name="c", subcore_axis_name="s")

    @functools.partial(
        pl.kernel,
        mesh=mesh,
        out_type=jax.ShapeDtypeStruct((VOCAB // 2, 2 * D_MODEL), jnp.float32),
        scratch_types=[
            *[pltpu.VMEM((D_MODEL, CH), jnp.float32) for _ in range(SRING)],
            *[pltpu.VMEM((D_MODEL, 2 * D_MODEL), jnp.float32) for _ in range(2)],
            *[pltpu.SemaphoreType.DMA for _ in range(SRING)],
            *[pltpu.SemaphoreType.DMA for _ in range(2)],
        ],
        compiler_params=_tc_tiled,
    )
    def k(lutT_hbm, tail2_hbm, t2_hbm, *scr):
        slabs = scr[:SRING]
        obufs = scr[SRING:SRING + 2]
        sems = scr[SRING + 2:2 * SRING + 2]
        osems = scr[2 * SRING + 2:]
        cid = lax.axis_index("c")
        sid = lax.axis_index("s")
        wid = cid * NS + sid

        for b in range(SRING - 1):
            s = wid + b * NW

            @pl.when(s < NSLAB)
            def _(b=b, s=s):
                pltpu.async_copy(lutT_hbm.at[:, pl.ds(s * CH, CH)], slabs[b], sems[b])

        riota = lax.iota(jnp.int32, LANES)

        def trip(g, carry):
            for b in range(SRING):
                n = g * SRING + b
                s = wid + n * NW

                @pl.when(s < NSLAB)
                def _(b=b, s=s, n=n):
                    s2 = s + (SRING - 1) * NW

                    @pl.when(s2 < NSLAB)
                    def _(b=b, s2=s2):
                        pltpu.async_copy(
                            lutT_hbm.at[:, pl.ds(s2 * CH, CH)],
                            slabs[(b + SRING - 1) % SRING],
                            sems[(b + SRING - 1) % SRING],
                        )

                    pltpu.make_async_copy(
                        lutT_hbm.at[:, pl.ds(s * CH, CH)], slabs[b], sems[b]
                    ).wait()

                    @pl.when(n >= 2)
                    def _(b=b, s=s):
                        pltpu.make_async_copy(
                            obufs[b % 2],
                            t2_hbm.at[pl.ds((s - 2 * NW) * D_MODEL, D_MODEL)],
                            osems[b % 2],
                        ).wait()

                    @plsc.parallel_loop(0, D_MODEL, 1, unroll=4)
                    def qloop(q, b=b):
                        c0 = 2 * q
                        for k0 in range(2 * D_MODEL // LANES):
                            col = c0 + (0 if k0 < D_MODEL // LANES else 1)
                            rvec = riota + (k0 % (D_MODEL // LANES)) * LANES
                            cvec = jnp.broadcast_to(col, (LANES,)).astype(jnp.int32)
                            v = plsc.load_gather(slabs[b], [rvec, cvec])
                            obufs[b % 2][q, pl.ds(k0 * LANES, LANES)] = v

                    pltpu.async_copy(
                        obufs[b % 2], t2_hbm.at[pl.ds(s * D_MODEL, D_MODEL)],
                        osems[b % 2],
                    )

            return carry

        lax.fori_loop(0, (SLAB_MAX + SRING - 1) // SRING, trip, 0)

        for b in range(2):
            @pl.when(wid + b * NW < NSLAB)
            def _(b=b):
                pltpu.make_async_copy(
                    obufs[b], t2_hbm.at[pl.ds(0, D_MODEL)], osems[b]
                ).wait()

        @pl.when(wid == NW - 1)
        def _():
            pltpu.sync_copy(tail2_hbm, obufs[0].at[pl.ds(0, TAIL_W // 2)])
            pltpu.sync_copy(
                obufs[0].at[pl.ds(0, TAIL_W // 2)],
                t2_hbm.at[pl.ds(TAIL_C0 // 2, TAIL_W // 2)],
            )

    return k(lutT, tail2)


def _sc_gather(xflat, t1m):
    mesh = plsc.VectorSubcoreMesh(core_axis_name="c", subcore_axis_name="s")

    @functools.partial(
        pl.kernel,
        mesh=mesh,
        out_type=jax.ShapeDtypeStruct(
            (NJ, D_MODEL // 8, NI // CH, 8, CH), jnp.float32
        ),
        scratch_types=[
            pltpu.VMEM((NIDX,), jnp.int32),
            *[pltpu.VMEM((CH, D_MODEL), jnp.float32) for _ in range(NBUF)],
            *[pltpu.VMEM((D_MODEL // 8, 1, 8, CH), jnp.float32) for _ in range(2)],
            *[pltpu.SemaphoreType.DMA for _ in range(NBUF)],
            *[pltpu.SemaphoreType.DMA for _ in range(2)],
        ],
        compiler_params=_linear,
    )
    def k(x_hbm, t_hbm, o4_hbm, idxall, *scr):
        gb = scr[:NBUF]
        tbufs = scr[NBUF:NBUF + 2]
        sems = scr[NBUF + 2:NBUF + 2 + NBUF]
        osems = scr[NBUF + 2 + NBUF:]

        cid = lax.axis_index("c")
        sid = lax.axis_index("s")
        wid = cid * NS + sid
        u0 = wid * PER_W

        pltpu.sync_copy(x_hbm.at[pl.ds(wid * NIDX, NIDX)], idxall)

        def fetch(t, b):
            pltpu.async_copy(
                t_hbm.at[idxall.at[pl.ds(t * CH, CH)]], gb[b], sems[b]
            )

        for b in range(NBUF):
            fetch(b, b)

        riota = lax.iota(jnp.int32, LANES)

        def step(g, carry):
            for b in range(NBUF):
                t = g * NBUF + b
                u = u0 + t
                j = lax.shift_right_logical(u, 5)
                ib = lax.bitwise_and(u, 31)
                tb = b % 2
                pltpu.make_async_copy(
                    t_hbm.at[idxall.at[pl.ds(t * CH, CH)]], gb[b], sems[b]
                ).wait()

                @pl.when(t >= 2)
                def _(tb=tb):
                    pltpu.make_async_copy(
                        tbufs[tb], o4_hbm.at[0, :, pl.ds(0, 1)], osems[tb]
                    ).wait()

                for q in range(CH // LANES):
                    r0 = q * LANES
                    rows = riota + r0

                    @plsc.parallel_loop(0, D_MODEL, 1, unroll=8)
                    def dloop(d, rows=rows, b=b, r0=r0, tb=tb):
                        cvec = jnp.broadcast_to(d, (LANES,)).astype(jnp.int32)
                        v = plsc.load_gather(gb[b], [rows, cvec]) * SCALE
                        tbufs[tb][
                            lax.shift_right_logical(d, 3), 0,
                            lax.bitwise_and(d, 7), pl.ds(r0, LANES)
                        ] = v

                pltpu.async_copy(
                    tbufs[tb], o4_hbm.at[j, :, pl.ds(ib, 1)], osems[tb]
                )

                nxt = t + NBUF

                @pl.when(nxt < PER_W)
                def _(b=b, nxt=nxt):
                    fetch(nxt, b)

            return carry

        lax.fori_loop(0, PER_W // NBUF, step, 0)

        for tb in range(2):
            pltpu.make_async_copy(
                tbufs[tb], o4_hbm.at[0, :, pl.ds(0, 1)], osems[tb]
            ).wait()

    return k(xflat, t1m)


def kernel(x, lut):
    xflat = x.astype(jnp.int32).T.reshape(-1)
    lutT = lut.T
    tail2 = lut[TAIL_C0:].reshape(TAIL_W // 2, 2 * D_MODEL)
    t2 = _sc_repack(lutT, tail2)
    t1m = t2.reshape(VOCAB, D_MODEL)
    o4 = _sc_gather(xflat, t1m)
    return o4.transpose(2, 4, 0, 1, 3).reshape(NI, NJ, D_MODEL)


# speedup vs baseline: 4.2766x; 2.0208x over previous
import functools
import math

import jax
import jax.numpy as jnp
from jax import lax
from jax.experimental import pallas as pl
from jax.experimental.pallas import tpu as pltpu
from jax.experimental.pallas import tpu_sc as plsc

D_MODEL = 64
VOCAB = 1000000
SCALE = math.sqrt(D_MODEL)

NC = 2
NS = 16
NW = NC * NS
LANES = 16

NI = 4096
NJ = 200
CH = 128
NUNITS = NJ * (NI // CH)
PER_W = NUNITS // NW
NIDX = PER_W * CH
NBUF = 4

NSLAB = VOCAB // CH
SLAB_MAX = NSLAB // NW + 1
SRING = 4
TAIL_C0 = NSLAB * CH
TAIL_W = VOCAB - TAIL_C0

_tc_tiled = pltpu.CompilerParams(use_tc_tiling_on_sc=True, needs_layout_passes=False)
_linear = pltpu.CompilerParams(use_tc_tiling_on_sc=False, needs_layout_passes=False)


def _sc_repack(lutT, tail2):
    mesh = plsc.VectorSubcoreMesh(core_axis_name="c", subcore_axis_name="s")

    @functools.partial(
        pl.kernel,
        mesh=mesh,
        out_type=jax.ShapeDtypeStruct((VOCAB // 2, 2 * D_MODEL), jnp.float32),
        scratch_types=[
            *[pltpu.VMEM((D_MODEL, CH), jnp.float32) for _ in range(SRING)],
            *[pltpu.VMEM((D_MODEL, 2 * D_MODEL), jnp.float32) for _ in range(2)],
            *[pltpu.SemaphoreType.DMA for _ in range(SRING)],
            *[pltpu.SemaphoreType.DMA for _ in range(2)],
        ],
        compiler_params=_tc_tiled,
    )
    def k(lutT_hbm, tail2_hbm, t2_hbm, *scr):
        slabs = scr[:SRING]
        obufs = scr[SRING:SRING + 2]
        sems = scr[SRING + 2:2 * SRING + 2]
        osems = scr[2 * SRING + 2:]
        cid = lax.axis_index("c")
        sid = lax.axis_index("s")
        wid = cid * NS + sid

        for b in range(SRING - 1):
            s = wid + b * NW

            @pl.when(s < NSLAB)
            def _(b=b, s=s):
                pltpu.async_copy(lutT_hbm.at[:, pl.ds(s * CH, CH)], slabs[b], sems[b])

        riota = lax.iota(jnp.int32, LANES)

        def trip(g, carry):
            for b in range(SRING):
                n = g * SRING + b
                s = wid + n * NW

                @pl.when(s < NSLAB)
                def _(b=b, s=s, n=n):
                    s2 = s + (SRING - 1) * NW

                    @pl.when(s2 < NSLAB)
                    def _(b=b, s2=s2):
                        pltpu.async_copy(
                            lutT_hbm.at[:, pl.ds(s2 * CH, CH)],
                            slabs[(b + SRING - 1) % SRING],
                            sems[(b + SRING - 1) % SRING],
                        )

                    pltpu.make_async_copy(
                        lutT_hbm.at[:, pl.ds(s * CH, CH)], slabs[b], sems[b]
                    ).wait()

                    @pl.when(n >= 2)
                    def _(b=b, s=s):
                        pltpu.make_async_copy(
                            obufs[b % 2],
                            t2_hbm.at[pl.ds((s - 2 * NW) * D_MODEL, D_MODEL)],
                            osems[b % 2],
                        ).wait()

                    for D0 in range(0, D_MODEL, LANES):
                        rowv = riota + D0

                        @plsc.parallel_loop(0, LANES, 1, unroll=4)
                        def kpass(kk, b=b, rowv=rowv, D0=D0):
                            diag = lax.bitwise_and(riota + kk, LANES - 1)
                            for C0 in range(0, CH, LANES):
                                ccv = diag + C0
                                v = plsc.load_gather(slabs[b], [rowv, ccv])
                                flat = lax.shift_left(ccv, 6) + rowv
                                qv = lax.shift_right_logical(flat, 7)
                                kv = lax.bitwise_and(flat, 127)
                                plsc.store_scatter(obufs[b % 2], [qv, kv], v)

                    pltpu.async_copy(
                        obufs[b % 2], t2_hbm.at[pl.ds(s * D_MODEL, D_MODEL)],
                        osems[b % 2],
                    )

            return carry

        lax.fori_loop(0, (SLAB_MAX + SRING - 1) // SRING, trip, 0)

        for b in range(2):
            @pl.when(wid + b * NW < NSLAB)
            def _(b=b):
                pltpu.make_async_copy(
                    obufs[b], t2_hbm.at[pl.ds(0, D_MODEL)], osems[b]
                ).wait()

        @pl.when(wid == NW - 1)
        def _():
            pltpu.sync_copy(tail2_hbm, obufs[0].at[pl.ds(0, TAIL_W // 2)])
            pltpu.sync_copy(
                obufs[0].at[pl.ds(0, TAIL_W // 2)],
                t2_hbm.at[pl.ds(TAIL_C0 // 2, TAIL_W // 2)],
            )

    return k(lutT, tail2)


def _sc_gather(xflat, t1m):
    mesh = plsc.VectorSubcoreMesh(core_axis_name="c", subcore_axis_name="s")

    @functools.partial(
        pl.kernel,
        mesh=mesh,
        out_type=jax.ShapeDtypeStruct(
            (NJ, D_MODEL // 8, NI // CH, 8, CH), jnp.float32
        ),
        scratch_types=[
            pltpu.VMEM((NIDX,), jnp.int32),
            *[pltpu.VMEM((CH, D_MODEL), jnp.float32) for _ in range(NBUF)],
            *[pltpu.VMEM((D_MODEL // 8, 1, 8, CH), jnp.float32) for _ in range(2)],
            *[pltpu.SemaphoreType.DMA for _ in range(NBUF)],
            *[pltpu.SemaphoreType.DMA for _ in range(2)],
        ],
        compiler_params=_linear,
    )
    def k(x_hbm, t_hbm, o4_hbm, idxall, *scr):
        gb = scr[:NBUF]
        tbufs = scr[NBUF:NBUF + 2]
        sems = scr[NBUF + 2:NBUF + 2 + NBUF]
        osems = scr[NBUF + 2 + NBUF:]

        cid = lax.axis_index("c")
        sid = lax.axis_index("s")
        wid = cid * NS + sid
        u0 = wid * PER_W

        pltpu.sync_copy(x_hbm.at[pl.ds(wid * NIDX, NIDX)], idxall)

        def fetch(t, b):
            pltpu.async_copy(
                t_hbm.at[idxall.at[pl.ds(t * CH, CH)]], gb[b], sems[b]
            )

        for b in range(NBUF):
            fetch(b, b)

        riota = lax.iota(jnp.int32, LANES)

        def step(g, carry):
            for b in range(NBUF):
                t = g * NBUF + b
                u = u0 + t
                j = lax.shift_right_logical(u, 5)
                ib = lax.bitwise_and(u, 31)
                tb = b % 2
                pltpu.make_async_copy(
                    t_hbm.at[idxall.at[pl.ds(t * CH, CH)]], gb[b], sems[b]
                ).wait()

                @pl.when(t >= 2)
                def _(tb=tb):
                    pltpu.make_async_copy(
                        tbufs[tb], o4_hbm.at[0, :, pl.ds(0, 1)], osems[tb]
                    ).wait()

                zero16 = jnp.zeros((LANES,), jnp.int32)
                for R0 in range(0, CH, LANES):
                    rows = riota + R0

                    @plsc.parallel_loop(0, LANES, 1, unroll=4)
                    def kpass(kk, rows=rows, b=b, tb=tb, zero16=zero16):
                        diag = lax.bitwise_and(riota + kk, LANES - 1)
                        for D0 in range(0, D_MODEL, LANES):
                            dv = diag + D0
                            v = plsc.load_gather(gb[b], [rows, dv]) * SCALE
                            av = lax.shift_right_logical(dv, 3)
                            cv = lax.bitwise_and(dv, 7)
                            plsc.store_scatter(tbufs[tb], [av, zero16, cv, rows], v)

                pltpu.async_copy(
                    tbufs[tb], o4_hbm.at[j, :, pl.ds(ib, 1)], osems[tb]
                )

                nxt = t + NBUF

                @pl.when(nxt < PER_W)
                def _(b=b, nxt=nxt):
                    fetch(nxt, b)

            return carry

        lax.fori_loop(0, PER_W // NBUF, step, 0)

        for tb in range(2):
            pltpu.make_async_copy(
                tbufs[tb], o4_hbm.at[0, :, pl.ds(0, 1)], osems[tb]
            ).wait()

    return k(xflat, t1m)


def kernel(x, lut):
    xflat = x.astype(jnp.int32).T.reshape(-1)
    lutT = lut.T
    tail2 = lut[TAIL_C0:].reshape(TAIL_W // 2, 2 * D_MODEL)
    t2 = _sc_repack(lutT, tail2)
    t1m = t2.reshape(VOCAB, D_MODEL)
    o4 = _sc_gather(xflat, t1m)
    return o4.transpose(2, 4, 0, 1, 3).reshape(NI, NJ, D_MODEL)


# speedup vs baseline: 6.3488x; 1.4846x over previous
import functools
import math

import jax
import jax.numpy as jnp
from jax import lax
from jax.experimental import pallas as pl
from jax.experimental.pallas import tpu as pltpu
from jax.experimental.pallas import tpu_sc as plsc

D_MODEL = 64
VOCAB = 1000000
SCALE = math.sqrt(D_MODEL)

NC = 2
NS = 16
NW = NC * NS
LANES = 16

NI = 4096
NJ = 200
CH = 128
NUNITS = NJ * (NI // CH)
PER_W = NUNITS // NW
NIDX = PER_W * CH
NBUF = 8

NSLAB = VOCAB // CH
SLAB_MAX = NSLAB // NW + 1
SRING = 4
TAIL_C0 = NSLAB * CH
TAIL_W = VOCAB - TAIL_C0

_tc_tiled = pltpu.CompilerParams(use_tc_tiling_on_sc=True, needs_layout_passes=False)
_linear = pltpu.CompilerParams(use_tc_tiling_on_sc=False, needs_layout_passes=False)


def _sc_repack(lutT, tail2):
    mesh = plsc.VectorSubcoreMesh(core_axis_name="c", subcore_axis_name="s")

    @functools.partial(
        pl.kernel,
        mesh=mesh,
        out_type=jax.ShapeDtypeStruct((VOCAB // 2, 2 * D_MODEL), jnp.float32),
        scratch_types=[
            *[pltpu.VMEM((D_MODEL, CH), jnp.float32) for _ in range(SRING)],
            *[pltpu.VMEM((D_MODEL, 2 * D_MODEL), jnp.float32) for _ in range(2)],
            *[pltpu.SemaphoreType.DMA for _ in range(SRING)],
            *[pltpu.SemaphoreType.DMA for _ in range(2)],
        ],
        compiler_params=_tc_tiled,
    )
    def k(lutT_hbm, tail2_hbm, t2_hbm, *scr):
        slabs = scr[:SRING]
        obufs = scr[SRING:SRING + 2]
        sems = scr[SRING + 2:2 * SRING + 2]
        osems = scr[2 * SRING + 2:]
        cid = lax.axis_index("c")
        sid = lax.axis_index("s")
        wid = cid * NS + sid

        for b in range(SRING - 1):
            s = wid + b * NW

            @pl.when(s < NSLAB)
            def _(b=b, s=s):
                pltpu.async_copy(lutT_hbm.at[:, pl.ds(s * CH, CH)], slabs[b], sems[b])

        riota = lax.iota(jnp.int32, LANES)

        def trip(g, carry):
            for b in range(SRING):
                n = g * SRING + b
                s = wid + n * NW

                @pl.when(s < NSLAB)
                def _(b=b, s=s, n=n):
                    s2 = s + (SRING - 1) * NW

                    @pl.when(s2 < NSLAB)
                    def _(b=b, s2=s2):
                        pltpu.async_copy(
                            lutT_hbm.at[:, pl.ds(s2 * CH, CH)],
                            slabs[(b + SRING - 1) % SRING],
                            sems[(b + SRING - 1) % SRING],
                        )

                    pltpu.make_async_copy(
                        lutT_hbm.at[:, pl.ds(s * CH, CH)], slabs[b], sems[b]
                    ).wait()

                    @pl.when(n >= 2)
                    def _(b=b, s=s):
                        pltpu.make_async_copy(
                            obufs[b % 2],
                            t2_hbm.at[pl.ds((s - 2 * NW) * D_MODEL, D_MODEL)],
                            osems[b % 2],
                        ).wait()

                    for D0 in range(0, D_MODEL, LANES):
                        rowv = riota + D0

                        @plsc.parallel_loop(0, LANES, 1, unroll=4)
                        def kpass(kk, b=b, rowv=rowv, D0=D0):
                            diag = lax.bitwise_and(riota + kk, LANES - 1)
                            for C0 in range(0, CH, LANES):
                                ccv = diag + C0
                                v = plsc.load_gather(slabs[b], [rowv, ccv])
                                flat = lax.shift_left(ccv, 6) + rowv
                                qv = lax.shift_right_logical(flat, 7)
                                kv = lax.bitwise_and(flat, 127)
                                plsc.store_scatter(obufs[b % 2], [qv, kv], v)

                    pltpu.async_copy(
                        obufs[b % 2], t2_hbm.at[pl.ds(s * D_MODEL, D_MODEL)],
                        osems[b % 2],
                    )

            return carry

        lax.fori_loop(0, (SLAB_MAX + SRING - 1) // SRING, trip, 0)

        for b in range(2):
            @pl.when(wid + b * NW < NSLAB)
            def _(b=b):
                pltpu.make_async_copy(
                    obufs[b], t2_hbm.at[pl.ds(0, D_MODEL)], osems[b]
                ).wait()

        @pl.when(wid == NW - 1)
        def _():
            pltpu.sync_copy(tail2_hbm, obufs[0].at[pl.ds(0, TAIL_W // 2)])
            pltpu.sync_copy(
                obufs[0].at[pl.ds(0, TAIL_W // 2)],
                t2_hbm.at[pl.ds(TAIL_C0 // 2, TAIL_W // 2)],
            )

    return k(lutT, tail2)


def _sc_gather(xflat, t1m):
    mesh = plsc.VectorSubcoreMesh(core_axis_name="c", subcore_axis_name="s")

    @functools.partial(
        pl.kernel,
        mesh=mesh,
        out_type=jax.ShapeDtypeStruct(
            (NJ, D_MODEL // 8, NI // CH, 8, CH), jnp.float32
        ),
        scratch_types=[
            pltpu.VMEM((NIDX,), jnp.int32),
            *[pltpu.VMEM((CH, D_MODEL), jnp.float32) for _ in range(NBUF)],
            *[pltpu.VMEM((D_MODEL // 8, 1, 8, CH), jnp.float32) for _ in range(2)],
            *[pltpu.SemaphoreType.DMA for _ in range(NBUF)],
            *[pltpu.SemaphoreType.DMA for _ in range(2)],
        ],
        compiler_params=_linear,
    )
    def k(x_hbm, t_hbm, o4_hbm, idxall, *scr):
        gb = scr[:NBUF]
        tbufs = scr[NBUF:NBUF + 2]
        sems = scr[NBUF + 2:NBUF + 2 + NBUF]
        osems = scr[NBUF + 2 + NBUF:]

        cid = lax.axis_index("c")
        sid = lax.axis_index("s")
        wid = cid * NS + sid
        u0 = wid * PER_W

        pltpu.sync_copy(x_hbm.at[pl.ds(wid * NIDX, NIDX)], idxall)

        def fetch(t, b):
            pltpu.async_copy(
                t_hbm.at[idxall.at[pl.ds(t * CH, CH)]], gb[b], sems[b]
            )

        for b in range(NBUF):
            fetch(b, b)

        riota = lax.iota(jnp.int32, LANES)

        def step(g, carry):
            for b in range(NBUF):
                t = g * NBUF + b
                u = u0 + t
                j = lax.shift_right_logical(u, 5)
                ib = lax.bitwise_and(u, 31)
                tb = b % 2
                pltpu.make_async_copy(
                    t_hbm.at[idxall.at[pl.ds(t * CH, CH)]], gb[b], sems[b]
                ).wait()

                @pl.when(t >= 2)
                def _(tb=tb):
                    pltpu.make_async_copy(
                        tbufs[tb], o4_hbm.at[0, :, pl.ds(0, 1)], osems[tb]
                    ).wait()

                zero16 = jnp.zeros((LANES,), jnp.int32)

                @plsc.parallel_loop(0, CH, 1, unroll=8)
                def kpass(m, b=b, tb=tb, zero16=zero16):
                    rows = riota + lax.bitwise_and(m, ~(LANES - 1))
                    diag = lax.bitwise_and(riota + m, LANES - 1)
                    for D0 in range(0, D_MODEL, LANES):
                        dv = diag + D0
                        v = plsc.load_gather(gb[b], [rows, dv]) * SCALE
                        av = lax.shift_right_logical(dv, 3)
                        cv = lax.bitwise_and(dv, 7)
                        plsc.store_scatter(tbufs[tb], [av, zero16, cv, rows], v)

                pltpu.async_copy(
                    tbufs[tb], o4_hbm.at[j, :, pl.ds(ib, 1)], osems[tb]
                )

                nxt = t + NBUF

                @pl.when(nxt < PER_W)
                def _(b=b, nxt=nxt):
                    fetch(nxt, b)

            return carry

        lax.fori_loop(0, PER_W // NBUF, step, 0)

        for tb in range(2):
            pltpu.make_async_copy(
                tbufs[tb], o4_hbm.at[0, :, pl.ds(0, 1)], osems[tb]
            ).wait()

    return k(xflat, t1m)


def kernel(x, lut):
    xflat = x.astype(jnp.int32).T.reshape(-1)
    lutT = lut.T
    tail2 = lut[TAIL_C0:].reshape(TAIL_W // 2, 2 * D_MODEL)
    t2 = _sc_repack(lutT, tail2)
    t1m = t2.reshape(VOCAB, D_MODEL)
    o4 = _sc_gather(xflat, t1m)
    return o4.transpose(2, 4, 0, 1, 3).reshape(NI, NJ, D_MODEL)


# speedup vs baseline: 6.9023x; 1.0872x over previous
import functools
import math

import jax
import jax.numpy as jnp
from jax import lax
from jax.experimental import pallas as pl
from jax.experimental.pallas import tpu as pltpu
from jax.experimental.pallas import tpu_sc as plsc

D_MODEL = 64
VOCAB = 1000000
SCALE = math.sqrt(D_MODEL)

NC = 2
NS = 16
NW = NC * NS
LANES = 16

NI = 4096
NJ = 200
CH = 128
NUNITS = NJ * (NI // CH)
PER_W = NUNITS // NW
NIDX = PER_W * CH
NBUF = 8

NSLAB = VOCAB // CH
SLAB_MAX = NSLAB // NW + 1
SRING = 6
TAIL_C0 = NSLAB * CH
TAIL_W = VOCAB - TAIL_C0

_tc_tiled = pltpu.CompilerParams(use_tc_tiling_on_sc=True, needs_layout_passes=False)
_linear = pltpu.CompilerParams(use_tc_tiling_on_sc=False, needs_layout_passes=False)


def _sc_repack(lutT, tail2):
    mesh = plsc.VectorSubcoreMesh(core_axis_name="c", subcore_axis_name="s")

    @functools.partial(
        pl.kernel,
        mesh=mesh,
        out_type=jax.ShapeDtypeStruct((VOCAB // 2, 2 * D_MODEL), jnp.float32),
        scratch_types=[
            *[pltpu.VMEM((D_MODEL, CH), jnp.float32) for _ in range(SRING)],
            *[pltpu.VMEM((D_MODEL, 2 * D_MODEL), jnp.float32) for _ in range(2)],
            *[pltpu.SemaphoreType.DMA for _ in range(SRING)],
            *[pltpu.SemaphoreType.DMA for _ in range(2)],
        ],
        compiler_params=_tc_tiled,
    )
    def k(lutT_hbm, tail2_hbm, t2_hbm, *scr):
        slabs = scr[:SRING]
        obufs = scr[SRING:SRING + 2]
        sems = scr[SRING + 2:2 * SRING + 2]
        osems = scr[2 * SRING + 2:]
        cid = lax.axis_index("c")
        sid = lax.axis_index("s")
        wid = cid * NS + sid

        for b in range(SRING - 1):
            s = wid + b * NW

            @pl.when(s < NSLAB)
            def _(b=b, s=s):
                pltpu.async_copy(lutT_hbm.at[:, pl.ds(s * CH, CH)], slabs[b], sems[b])

        riota = lax.iota(jnp.int32, LANES)

        def trip(g, carry):
            for b in range(SRING):
                n = g * SRING + b
                s = wid + n * NW

                @pl.when(s < NSLAB)
                def _(b=b, s=s, n=n):
                    s2 = s + (SRING - 1) * NW

                    @pl.when(s2 < NSLAB)
                    def _(b=b, s2=s2):
                        pltpu.async_copy(
                            lutT_hbm.at[:, pl.ds(s2 * CH, CH)],
                            slabs[(b + SRING - 1) % SRING],
                            sems[(b + SRING - 1) % SRING],
                        )

                    pltpu.make_async_copy(
                        lutT_hbm.at[:, pl.ds(s * CH, CH)], slabs[b], sems[b]
                    ).wait()

                    @pl.when(n >= 2)
                    def _(b=b, s=s):
                        pltpu.make_async_copy(
                            obufs[b % 2],
                            t2_hbm.at[pl.ds((s - 2 * NW) * D_MODEL, D_MODEL)],
                            osems[b % 2],
                        ).wait()

                    @plsc.parallel_loop(0, D_MODEL, 1, unroll=8)
                    def kpass(m, b=b):
                        rowv = riota + lax.bitwise_and(m, ~(LANES - 1))
                        diag = lax.bitwise_and(riota + m, LANES - 1)
                        for C0 in range(0, CH, LANES):
                            ccv = diag + C0
                            v = plsc.load_gather(slabs[b], [rowv, ccv])
                            flat = lax.shift_left(ccv, 6) + rowv
                            qv = lax.shift_right_logical(flat, 7)
                            kv = lax.bitwise_and(flat, 127)
                            plsc.store_scatter(obufs[b % 2], [qv, kv], v)

                    pltpu.async_copy(
                        obufs[b % 2], t2_hbm.at[pl.ds(s * D_MODEL, D_MODEL)],
                        osems[b % 2],
                    )

            return carry

        lax.fori_loop(0, (SLAB_MAX + SRING - 1) // SRING, trip, 0)

        for b in range(2):
            @pl.when(wid + b * NW < NSLAB)
            def _(b=b):
                pltpu.make_async_copy(
                    obufs[b], t2_hbm.at[pl.ds(0, D_MODEL)], osems[b]
                ).wait()

        @pl.when(wid == NW - 1)
        def _():
            pltpu.sync_copy(tail2_hbm, obufs[0].at[pl.ds(0, TAIL_W // 2)])
            pltpu.sync_copy(
                obufs[0].at[pl.ds(0, TAIL_W // 2)],
                t2_hbm.at[pl.ds(TAIL_C0 // 2, TAIL_W // 2)],
            )

    return k(lutT, tail2)


def _sc_gather(xflat, t1m):
    mesh = plsc.VectorSubcoreMesh(core_axis_name="c", subcore_axis_name="s")

    @functools.partial(
        pl.kernel,
        mesh=mesh,
        out_type=jax.ShapeDtypeStruct(
            (NJ, D_MODEL // 8, NI // CH, 8, CH), jnp.float32
        ),
        scratch_types=[
            pltpu.VMEM((NIDX,), jnp.int32),
            *[pltpu.VMEM((CH, D_MODEL), jnp.float32) for _ in range(NBUF)],
            *[pltpu.VMEM((D_MODEL // 8, 1, 8, CH), jnp.float32) for _ in range(2)],
            *[pltpu.SemaphoreType.DMA for _ in range(NBUF)],
            *[pltpu.SemaphoreType.DMA for _ in range(2)],
        ],
        compiler_params=_linear,
    )
    def k(x_hbm, t_hbm, o4_hbm, idxall, *scr):
        gb = scr[:NBUF]
        tbufs = scr[NBUF:NBUF + 2]
        sems = scr[NBUF + 2:NBUF + 2 + NBUF]
        osems = scr[NBUF + 2 + NBUF:]

        cid = lax.axis_index("c")
        sid = lax.axis_index("s")
        wid = cid * NS + sid
        u0 = wid * PER_W

        pltpu.sync_copy(x_hbm.at[pl.ds(wid * NIDX, NIDX)], idxall)

        def fetch(t, b):
            pltpu.async_copy(
                t_hbm.at[idxall.at[pl.ds(t * CH, CH)]], gb[b], sems[b]
            )

        for b in range(NBUF):
            fetch(b, b)

        riota = lax.iota(jnp.int32, LANES)

        def step(g, carry):
            for b in range(NBUF):
                t = g * NBUF + b
                u = u0 + t
                j = lax.shift_right_logical(u, 5)
                ib = lax.bitwise_and(u, 31)
                tb = b % 2
                pltpu.make_async_copy(
                    t_hbm.at[idxall.at[pl.ds(t * CH, CH)]], gb[b], sems[b]
                ).wait()

                @pl.when(t >= 2)
                def _(tb=tb):
                    pltpu.make_async_copy(
                        tbufs[tb], o4_hbm.at[0, :, pl.ds(0, 1)], osems[tb]
                    ).wait()

                zero16 = jnp.zeros((LANES,), jnp.int32)

                @plsc.parallel_loop(0, CH, 1, unroll=8)
                def kpass(m, b=b, tb=tb, zero16=zero16):
                    rows = riota + lax.bitwise_and(m, ~(LANES - 1))
                    diag = lax.bitwise_and(riota + m, LANES - 1)
                    for D0 in range(0, D_MODEL, LANES):
                        dv = diag + D0
                        v = plsc.load_gather(gb[b], [rows, dv]) * SCALE
                        av = lax.shift_right_logical(dv, 3)
                        cv = lax.bitwise_and(dv, 7)
                        plsc.store_scatter(tbufs[tb], [av, zero16, cv, rows], v)

                pltpu.async_copy(
                    tbufs[tb], o4_hbm.at[j, :, pl.ds(ib, 1)], osems[tb]
                )

                nxt = t + NBUF

                @pl.when(nxt < PER_W)
                def _(b=b, nxt=nxt):
                    fetch(nxt, b)

            return carry

        lax.fori_loop(0, PER_W // NBUF, step, 0)

        for tb in range(2):
            pltpu.make_async_copy(
                tbufs[tb], o4_hbm.at[0, :, pl.ds(0, 1)], osems[tb]
            ).wait()

    return k(xflat, t1m)


def kernel(x, lut):
    xflat = x.astype(jnp.int32).T.reshape(-1)
    lutT = lut.T
    tail2 = lut[TAIL_C0:].reshape(TAIL_W // 2, 2 * D_MODEL)
    t2 = _sc_repack(lutT, tail2)
    t1m = t2.reshape(VOCAB, D_MODEL)
    o4 = _sc_gather(xflat, t1m)
    return o4.transpose(2, 4, 0, 1, 3).reshape(NI, NJ, D_MODEL)
